# Initial kernel scaffold; baseline (speedup 1.0000x reference)
import jax
import jax.numpy as jnp
from jax.experimental import pallas as pl


def kernel(x, a, e, Ws, bs, Wai, bai, Waj, baj, Wn, bn, We, be, Wd, bd):
    raise NotImplementedError("write your pallas kernel here")


# speedup vs baseline: 2.2026x; 2.2026x over previous
import jax
import jax.numpy as jnp
from jax import lax
from jax.experimental import pallas as pl
from jax.experimental.pallas import tpu as pltpu

_TI = 16


def _net_body(x_ref, a_ref, e_ref, et_ref, wsi_ref, wsj_ref, wvec_ref,
              bscal_ref, wnx_ref, wni_ref, wnj_ref, bn_ref, wd_ref, bd_ref,
              out_ref, pi_scr, u_scr):
    N = a_ref.shape[1]
    C = wvec_ref.shape[1]

    xb = x_ref[0]
    wvec = wvec_ref[...]
    we_ = wvec[0:1, :][None]
    wet_ = wvec[1:2, :][None]
    wai = wvec[3:4, :][None]
    waj = wvec[4:5, :][None]
    bai = bscal_ref[0]
    baj = bscal_ref[1]

    u_scr[...] = jnp.dot(xb, wsi_ref[...], preferred_element_type=jnp.float32) \
        + wvec[2:3, :]
    v = jnp.dot(xb, wsj_ref[...], preferred_element_type=jnp.float32)

    def body(t, pj):
        i0 = t * _TI
        ui = u_scr[pl.ds(i0, _TI), :]
        ei = e_ref[0, pl.ds(i0, _TI), :]
        eti = et_ref[0, pl.ds(i0, _TI), :]
        ai = a_ref[0, pl.ds(i0, _TI), :]
        pre = (ui[:, None, :] + v[None, :, :]
               + ei[:, :, None] * we_ + eti[:, :, None] * wet_)
        s = jnp.maximum(pre, 0.0) * ai[:, :, None]
        di = jnp.sum(s * wai, axis=2, keepdims=True) + bai
        dj = jnp.sum(s * waj, axis=2, keepdims=True) + baj
        pi = jnp.sum(s * jax.nn.sigmoid(di), axis=1)
        pi_scr[pl.ds(i0, _TI), :] = pi
        return pj + jnp.sum(s * jax.nn.sigmoid(dj), axis=0)

    pj = lax.fori_loop(0, N // _TI, body, jnp.zeros((N, C), jnp.float32))

    xo = (jnp.dot(xb, wnx_ref[...], preferred_element_type=jnp.float32)
          + jnp.dot(pi_scr[...], wni_ref[...],
                    preferred_element_type=jnp.float32)
          + jnp.dot(pj, wnj_ref[...], preferred_element_type=jnp.float32)
          + bn_ref[...])
    out_ref[0] = jnp.dot(xo, wd_ref[...],
                         preferred_element_type=jnp.float32) + bd_ref[...]


def kernel(x, a, e, Ws, bs, Wai, bai, Waj, baj, Wn, bn, We, be, Wd, bd):
    B, N, F = x.shape
    C = Ws.shape[1]
    LBL = Wd.shape[1]

    e2 = e[..., 0]
    et2 = jnp.swapaxes(e2, 1, 2)
    wsi = Ws[:F]
    wsj = Ws[F:2 * F]
    wvec = jnp.stack([Ws[2 * F], Ws[2 * F + 1], bs, Wai[:, 0], Waj[:, 0]],
                     axis=0)
    bscal = jnp.stack([bai[0], baj[0]])
    wnx = Wn[:F]
    wni = Wn[F:F + C]
    wnj = Wn[F + C:]

    out = pl.pallas_call(
        _net_body,
        grid=(B,),
        in_specs=[
            pl.BlockSpec((1, N, F), lambda b: (b, 0, 0)),
            pl.BlockSpec((1, N, N), lambda b: (b, 0, 0)),
            pl.BlockSpec((1, N, N), lambda b: (b, 0, 0)),
            pl.BlockSpec((1, N, N), lambda b: (b, 0, 0)),
            pl.BlockSpec((F, C), lambda b: (0, 0)),
            pl.BlockSpec((F, C), lambda b: (0, 0)),
            pl.BlockSpec((5, C), lambda b: (0, 0)),
            pl.BlockSpec(memory_space=pltpu.SMEM),
            pl.BlockSpec((F, F), lambda b: (0, 0)),
            pl.BlockSpec((C, F), lambda b: (0, 0)),
            pl.BlockSpec((C, F), lambda b: (0, 0)),
            pl.BlockSpec((1, F), lambda b: (0, 0)),
            pl.BlockSpec((F, LBL), lambda b: (0, 0)),
            pl.BlockSpec((1, LBL), lambda b: (0, 0)),
        ],
        out_specs=pl.BlockSpec((1, N, LBL), lambda b: (b, 0, 0)),
        out_shape=jax.ShapeDtypeStruct((B, N, LBL), jnp.float32),
        scratch_shapes=[pltpu.VMEM((N, C), jnp.float32),
                        pltpu.VMEM((N, C), jnp.float32)],
    )(x, a, e2, et2, wsi, wsj, wvec, bscal, wnx, wni, wnj,
      bn[None], Wd, bd[None])
    return out


# speedup vs baseline: 6.5954x; 2.9944x over previous
import jax
import jax.numpy as jnp
from jax import lax
from jax.experimental import pallas as pl
from jax.experimental.pallas import tpu as pltpu


def _net_body(x_ref, a_ref, e_ref, et_ref, wsi_ref, wsj_ref, bs_ref,
              wsc_ref, wnx_ref, wni_ref, wnj_ref, bn_ref, wd_ref, bd_ref,
              out_ref, s_scr, vt_scr, u_scr, di_scr, dj_scr, pi_scr, pjt_scr):
    N = a_ref.shape[1]
    C = bs_ref.shape[1]
    f32 = jnp.float32

    xb = x_ref[0]
    eb = e_ref[0]
    etb = et_ref[0]
    ab = a_ref[0]

    u_scr[...] = jnp.dot(xb, wsi_ref[...],
                         preferred_element_type=f32) + bs_ref[...]
    v = jnp.dot(xb, wsj_ref[...], preferred_element_type=f32)
    vt_scr[...] = v.T

    ones_col = jnp.ones((N, 1), f32)
    ones_row = jnp.ones((1, N), f32)
    iota_col = lax.broadcasted_iota(jnp.int32, (C, 1), 0)
    iota_row = lax.broadcasted_iota(jnp.int32, (1, C), 1)

    di_scr[...] = jnp.zeros((N, N), f32)
    dj_scr[...] = jnp.zeros((N, N), f32)

    def pass1(c, _):
        we_c = wsc_ref[0, c]
        wet_c = wsc_ref[1, c]
        wai_c = wsc_ref[2, c]
        waj_c = wsc_ref[3, c]
        oh_col = (iota_col == c).astype(f32)
        u_col = jnp.dot(u_scr[...], oh_col, preferred_element_type=f32)
        v_row = vt_scr[pl.ds(c, 1), :]
        lhs = jnp.concatenate([u_col, ones_col], axis=1)
        rhs = jnp.concatenate([ones_row, v_row], axis=0)
        uv = jnp.dot(lhs, rhs, preferred_element_type=f32)
        pre = uv + we_c * eb + wet_c * etb
        s_c = jnp.maximum(pre, 0.0) * ab
        s_scr[pl.ds(c, 1), :, :] = s_c[None]
        di_scr[...] = di_scr[...] + wai_c * s_c
        dj_scr[...] = dj_scr[...] + waj_c * s_c
        return 0

    lax.fori_loop(0, C, pass1, 0)

    bai = wsc_ref[4, 0]
    baj = wsc_ref[4, 1]
    sig_i = jax.nn.sigmoid(di_scr[...] + bai)
    sig_j = jax.nn.sigmoid(dj_scr[...] + baj)
    di_scr[...] = sig_i
    dj_scr[...] = sig_j

    pi_scr[...] = jnp.zeros((N, C), f32)
    pjt_scr[...] = jnp.zeros((C, N), f32)

    def pass2(c, _):
        oh_col = (iota_col == c).astype(f32)
        oh_row = (iota_row == c).astype(f32)
        s_c = s_scr[c]
        ti = s_c * di_scr[...]
        tj = s_c * dj_scr[...]
        pcol = jnp.dot(ti, ones_col, preferred_element_type=f32)
        prow = jnp.dot(ones_row, tj, preferred_element_type=f32)
        pi_scr[...] = pi_scr[...] + jnp.dot(pcol, oh_row,
                                            preferred_element_type=f32)
        pjt_scr[...] = pjt_scr[...] + jnp.dot(oh_col, prow,
                                              preferred_element_type=f32)
        return 0

    lax.fori_loop(0, C, pass2, 0)

    xo = (jnp.dot(xb, wnx_ref[...], preferred_element_type=f32)
          + jnp.dot(pi_scr[...], wni_ref[...], preferred_element_type=f32)
          + lax.dot_general(pjt_scr[...], wnj_ref[...],
                            (((0,), (0,)), ((), ())),
                            preferred_element_type=f32)
          + bn_ref[...])
    out_ref[0] = jnp.dot(xo, wd_ref[...], preferred_element_type=f32) \
        + bd_ref[...]


def kernel(x, a, e, Ws, bs, Wai, bai, Waj, baj, Wn, bn, We, be, Wd, bd):
    B, N, F = x.shape
    C = Ws.shape[1]
    LBL = Wd.shape[1]
    f32 = jnp.float32

    e2 = e[..., 0]
    et2 = jnp.swapaxes(e2, 1, 2)
    wsi = Ws[:F]
    wsj = Ws[F:2 * F]
    brow = jnp.zeros((C,), f32).at[0].set(bai[0]).at[1].set(baj[0])
    wsc = jnp.stack([Ws[2 * F], Ws[2 * F + 1], Wai[:, 0], Waj[:, 0], brow],
                    axis=0)
    wnx = Wn[:F]
    wni = Wn[F:F + C]
    wnj = Wn[F + C:]

    out = pl.pallas_call(
        _net_body,
        grid=(B,),
        in_specs=[
            pl.BlockSpec((1, N, F), lambda b: (b, 0, 0)),
            pl.BlockSpec((1, N, N), lambda b: (b, 0, 0)),
            pl.BlockSpec((1, N, N), lambda b: (b, 0, 0)),
            pl.BlockSpec((1, N, N), lambda b: (b, 0, 0)),
            pl.BlockSpec((F, C), lambda b: (0, 0)),
            pl.BlockSpec((F, C), lambda b: (0, 0)),
            pl.BlockSpec((1, C), lambda b: (0, 0)),
            pl.BlockSpec(memory_space=pltpu.SMEM),
            pl.BlockSpec((F, F), lambda b: (0, 0)),
            pl.BlockSpec((C, F), lambda b: (0, 0)),
            pl.BlockSpec((C, F), lambda b: (0, 0)),
            pl.BlockSpec((1, F), lambda b: (0, 0)),
            pl.BlockSpec((F, LBL), lambda b: (0, 0)),
            pl.BlockSpec((1, LBL), lambda b: (0, 0)),
        ],
        out_specs=pl.BlockSpec((1, N, LBL), lambda b: (b, 0, 0)),
        out_shape=jax.ShapeDtypeStruct((B, N, LBL), f32),
        scratch_shapes=[
            pltpu.VMEM((C, N, N), f32),
            pltpu.VMEM((C, N), f32),
            pltpu.VMEM((N, C), f32),
            pltpu.VMEM((N, N), f32),
            pltpu.VMEM((N, N), f32),
            pltpu.VMEM((N, C), f32),
            pltpu.VMEM((C, N), f32),
        ],
    )(x, a, e2, et2, wsi, wsj, bs[None], wsc, wnx, wni, wnj,
      bn[None], Wd, bd[None])
    return out


# speedup vs baseline: 8.9023x; 1.3498x over previous
import jax
import jax.numpy as jnp
from jax import lax
from jax.experimental import pallas as pl
from jax.experimental.pallas import tpu as pltpu


def _net_body(x_ref, a_ref, e_ref, et_ref, wsi_ref, wsj_ref, bs_ref,
              wsc_ref, wnx_ref, wni_ref, wnj_ref, bn_ref, wd_ref, bd_ref,
              out_ref, s_scr, vt_scr, u_scr, di_scr, dj_scr, pi_scr, pjt_scr):
    N = a_ref.shape[1]
    C = bs_ref.shape[1]
    f32 = jnp.float32

    xb = x_ref[0]

    u_scr[...] = jnp.dot(xb, wsi_ref[...],
                         preferred_element_type=f32) + bs_ref[...]
    v = jnp.dot(xb, wsj_ref[...], preferred_element_type=f32)
    vt_scr[...] = v.T

    ones_col = jnp.ones((N, 1), f32)
    ones_row = jnp.ones((1, N), f32)
    iota_cc = lax.broadcasted_iota(jnp.int32, (C, 2), 0)
    iota_2c = lax.broadcasted_iota(jnp.int32, (2, C), 1)
    two_col = lax.broadcasted_iota(jnp.int32, (C, 2), 1)
    two_row = lax.broadcasted_iota(jnp.int32, (2, C), 0)

    di_scr[...] = jnp.zeros((N, N), f32)
    dj_scr[...] = jnp.zeros((N, N), f32)

    def pass1(t, _):
        c0 = 2 * t
        eb = e_ref[0]
        etb = et_ref[0]
        ab = a_ref[0]
        oh2 = (iota_cc == c0 + two_col).astype(f32)
        ucols = jnp.dot(u_scr[...], oh2, preferred_element_type=f32)
        ub0 = jnp.broadcast_to(ucols[:, 0:1], (N, N))
        ub1 = jnp.broadcast_to(ucols[:, 1:2], (N, N))
        vb0 = jnp.broadcast_to(vt_scr[pl.ds(c0, 1), :], (N, N))
        vb1 = jnp.broadcast_to(vt_scr[pl.ds(c0 + 1, 1), :], (N, N))
        s0 = jnp.maximum(ub0 + vb0 + wsc_ref[0, c0] * eb
                         + wsc_ref[1, c0] * etb, 0.0) * ab
        s1 = jnp.maximum(ub1 + vb1 + wsc_ref[0, c0 + 1] * eb
                         + wsc_ref[1, c0 + 1] * etb, 0.0) * ab
        s_scr[pl.ds(c0, 1), :, :] = s0[None]
        s_scr[pl.ds(c0 + 1, 1), :, :] = s1[None]
        di_scr[...] = di_scr[...] + (wsc_ref[2, c0] * s0
                                     + wsc_ref[2, c0 + 1] * s1)
        dj_scr[...] = dj_scr[...] + (wsc_ref[3, c0] * s0
                                     + wsc_ref[3, c0 + 1] * s1)
        return 0

    lax.fori_loop(0, C // 2, pass1, 0)

    bai = wsc_ref[4, 0]
    baj = wsc_ref[4, 1]
    di_scr[...] = jax.nn.sigmoid(di_scr[...] + bai)
    dj_scr[...] = jax.nn.sigmoid(dj_scr[...] + baj)

    pi_scr[...] = jnp.zeros((N, C), f32)
    pjt_scr[...] = jnp.zeros((C, N), f32)

    def pass2(t, _):
        c0 = 2 * t
        sigi = di_scr[...]
        sigj = dj_scr[...]
        s0 = s_scr[c0]
        s1 = s_scr[c0 + 1]
        pcols = jnp.concatenate(
            [jnp.dot(s0 * sigi, ones_col, preferred_element_type=f32),
             jnp.dot(s1 * sigi, ones_col, preferred_element_type=f32)],
            axis=1)
        prows = jnp.concatenate(
            [jnp.dot(ones_row, s0 * sigj, preferred_element_type=f32),
             jnp.dot(ones_row, s1 * sigj, preferred_element_type=f32)],
            axis=0)
        ohrows = (iota_2c == c0 + two_row).astype(f32)
        pi_scr[...] = pi_scr[...] + jnp.dot(pcols, ohrows,
                                            preferred_element_type=f32)
        pjt_scr[...] = pjt_scr[...] + lax.dot_general(
            ohrows, prows, (((0,), (0,)), ((), ())),
            preferred_element_type=f32)
        return 0

    lax.fori_loop(0, C // 2, pass2, 0)

    xo = (jnp.dot(xb, wnx_ref[...], preferred_element_type=f32)
          + jnp.dot(pi_scr[...], wni_ref[...], preferred_element_type=f32)
          + lax.dot_general(pjt_scr[...], wnj_ref[...],
                            (((0,), (0,)), ((), ())),
                            preferred_element_type=f32)
          + bn_ref[...])
    out_ref[0] = jnp.dot(xo, wd_ref[...], preferred_element_type=f32) \
        + bd_ref[...]


def kernel(x, a, e, Ws, bs, Wai, bai, Waj, baj, Wn, bn, We, be, Wd, bd):
    B, N, F = x.shape
    C = Ws.shape[1]
    LBL = Wd.shape[1]
    f32 = jnp.float32

    e2 = e[..., 0]
    et2 = jnp.swapaxes(e2, 1, 2)
    wsi = Ws[:F]
    wsj = Ws[F:2 * F]
    brow = jnp.zeros((C,), f32).at[0].set(bai[0]).at[1].set(baj[0])
    wsc = jnp.stack([Ws[2 * F], Ws[2 * F + 1], Wai[:, 0], Waj[:, 0], brow],
                    axis=0)
    wnx = Wn[:F]
    wni = Wn[F:F + C]
    wnj = Wn[F + C:]

    out = pl.pallas_call(
        _net_body,
        grid=(B,),
        in_specs=[
            pl.BlockSpec((1, N, F), lambda b: (b, 0, 0)),
            pl.BlockSpec((1, N, N), lambda b: (b, 0, 0)),
            pl.BlockSpec((1, N, N), lambda b: (b, 0, 0)),
            pl.BlockSpec((1, N, N), lambda b: (b, 0, 0)),
            pl.BlockSpec((F, C), lambda b: (0, 0)),
            pl.BlockSpec((F, C), lambda b: (0, 0)),
            pl.BlockSpec((1, C), lambda b: (0, 0)),
            pl.BlockSpec(memory_space=pltpu.SMEM),
            pl.BlockSpec((F, F), lambda b: (0, 0)),
            pl.BlockSpec((C, F), lambda b: (0, 0)),
            pl.BlockSpec((C, F), lambda b: (0, 0)),
            pl.BlockSpec((1, F), lambda b: (0, 0)),
            pl.BlockSpec((F, LBL), lambda b: (0, 0)),
            pl.BlockSpec((1, LBL), lambda b: (0, 0)),
        ],
        out_specs=pl.BlockSpec((1, N, LBL), lambda b: (b, 0, 0)),
        out_shape=jax.ShapeDtypeStruct((B, N, LBL), f32),
        scratch_shapes=[
            pltpu.VMEM((C, N, N), f32),
            pltpu.VMEM((C, N), f32),
            pltpu.VMEM((N, C), f32),
            pltpu.VMEM((N, N), f32),
            pltpu.VMEM((N, N), f32),
            pltpu.VMEM((N, C), f32),
            pltpu.VMEM((C, N), f32),
        ],
    )(x, a, e2, et2, wsi, wsj, bs[None], wsc, wnx, wni, wnj,
      bn[None], Wd, bd[None])
    return out


# speedup vs baseline: 10.0920x; 1.1336x over previous
import jax
import jax.numpy as jnp
from jax import lax
from jax.experimental import pallas as pl
from jax.experimental.pallas import tpu as pltpu


def _net_body(x_ref, a_ref, e_ref, et_ref, wsi_ref, wsj_ref, bs_ref,
              wsc_ref, wnx_ref, wni_ref, wnj_ref, bn_ref, wd_ref, bd_ref,
              out_ref, s_scr, vt_scr, u_scr, di_scr, dj_scr, pi_scr, pjt_scr):
    N = a_ref.shape[1]
    C = bs_ref.shape[1]
    f32 = jnp.float32

    xb = x_ref[0]

    u_scr[...] = jnp.dot(xb, wsi_ref[...],
                         preferred_element_type=f32) + bs_ref[...]
    v = jnp.dot(xb, wsj_ref[...], preferred_element_type=f32)
    vt_scr[...] = v.T

    ones_col = jnp.ones((N, 1), f32)
    ones_row = jnp.ones((1, N), f32)
    iota_cc = lax.broadcasted_iota(jnp.int32, (C, 4), 0)
    iota_4c = lax.broadcasted_iota(jnp.int32, (4, C), 1)
    four_col = lax.broadcasted_iota(jnp.int32, (C, 4), 1)
    four_row = lax.broadcasted_iota(jnp.int32, (4, C), 0)

    di_scr[...] = jnp.zeros((N, N), f32)
    dj_scr[...] = jnp.zeros((N, N), f32)

    def pass1(t, _):
        c0 = 4 * t
        eb = e_ref[0]
        etb = et_ref[0]
        ab = a_ref[0]
        oh4 = (iota_cc == c0 + four_col).astype(f32)
        ucols = jnp.dot(u_scr[...], oh4, preferred_element_type=f32)
        sv = []
        for k in range(4):
            ub = jnp.broadcast_to(ucols[:, k:k + 1], (N, N))
            vb = jnp.broadcast_to(vt_scr[pl.ds(c0 + k, 1), :], (N, N))
            s_k = jnp.maximum(ub + vb + wsc_ref[0, c0 + k] * eb
                              + wsc_ref[1, c0 + k] * etb, 0.0) * ab
            s_scr[pl.ds(c0 + k, 1), :, :] = s_k[None]
            sv.append(s_k)
        di_scr[...] = di_scr[...] + (
            (wsc_ref[2, c0] * sv[0] + wsc_ref[2, c0 + 1] * sv[1])
            + (wsc_ref[2, c0 + 2] * sv[2] + wsc_ref[2, c0 + 3] * sv[3]))
        dj_scr[...] = dj_scr[...] + (
            (wsc_ref[3, c0] * sv[0] + wsc_ref[3, c0 + 1] * sv[1])
            + (wsc_ref[3, c0 + 2] * sv[2] + wsc_ref[3, c0 + 3] * sv[3]))
        return 0

    lax.fori_loop(0, C // 4, pass1, 0)

    bai = wsc_ref[4, 0]
    baj = wsc_ref[4, 1]
    di_scr[...] = jax.nn.sigmoid(di_scr[...] + bai)
    dj_scr[...] = jax.nn.sigmoid(dj_scr[...] + baj)

    pi_scr[...] = jnp.zeros((N, C), f32)
    pjt_scr[...] = jnp.zeros((C, N), f32)

    def pass2(t, _):
        c0 = 4 * t
        sigi = di_scr[...]
        sigj = dj_scr[...]
        pcl = []
        prl = []
        for k in range(4):
            s_k = s_scr[c0 + k]
            pcl.append(jnp.dot(s_k * sigi, ones_col,
                               preferred_element_type=f32))
            prl.append(jnp.dot(ones_row, s_k * sigj,
                               preferred_element_type=f32))
        pcols = jnp.concatenate(pcl, axis=1)
        prows = jnp.concatenate(prl, axis=0)
        ohrows = (iota_4c == c0 + four_row).astype(f32)
        pi_scr[...] = pi_scr[...] + jnp.dot(pcols, ohrows,
                                            preferred_element_type=f32)
        pjt_scr[...] = pjt_scr[...] + lax.dot_general(
            ohrows, prows, (((0,), (0,)), ((), ())),
            preferred_element_type=f32)
        return 0

    lax.fori_loop(0, C // 4, pass2, 0)

    xo = (jnp.dot(xb, wnx_ref[...], preferred_element_type=f32)
          + jnp.dot(pi_scr[...], wni_ref[...], preferred_element_type=f32)
          + lax.dot_general(pjt_scr[...], wnj_ref[...],
                            (((0,), (0,)), ((), ())),
                            preferred_element_type=f32)
          + bn_ref[...])
    out_ref[0] = jnp.dot(xo, wd_ref[...], preferred_element_type=f32) \
        + bd_ref[...]


def kernel(x, a, e, Ws, bs, Wai, bai, Waj, baj, Wn, bn, We, be, Wd, bd):
    B, N, F = x.shape
    C = Ws.shape[1]
    LBL = Wd.shape[1]
    f32 = jnp.float32

    e2 = e[..., 0]
    et2 = jnp.swapaxes(e2, 1, 2)
    wsi = Ws[:F]
    wsj = Ws[F:2 * F]
    brow = jnp.zeros((C,), f32).at[0].set(bai[0]).at[1].set(baj[0])
    wsc = jnp.stack([Ws[2 * F], Ws[2 * F + 1], Wai[:, 0], Waj[:, 0], brow],
                    axis=0)
    wnx = Wn[:F]
    wni = Wn[F:F + C]
    wnj = Wn[F + C:]

    out = pl.pallas_call(
        _net_body,
        grid=(B,),
        in_specs=[
            pl.BlockSpec((1, N, F), lambda b: (b, 0, 0)),
            pl.BlockSpec((1, N, N), lambda b: (b, 0, 0)),
            pl.BlockSpec((1, N, N), lambda b: (b, 0, 0)),
            pl.BlockSpec((1, N, N), lambda b: (b, 0, 0)),
            pl.BlockSpec((F, C), lambda b: (0, 0)),
            pl.BlockSpec((F, C), lambda b: (0, 0)),
            pl.BlockSpec((1, C), lambda b: (0, 0)),
            pl.BlockSpec(memory_space=pltpu.SMEM),
            pl.BlockSpec((F, F), lambda b: (0, 0)),
            pl.BlockSpec((C, F), lambda b: (0, 0)),
            pl.BlockSpec((C, F), lambda b: (0, 0)),
            pl.BlockSpec((1, F), lambda b: (0, 0)),
            pl.BlockSpec((F, LBL), lambda b: (0, 0)),
            pl.BlockSpec((1, LBL), lambda b: (0, 0)),
        ],
        out_specs=pl.BlockSpec((1, N, LBL), lambda b: (b, 0, 0)),
        out_shape=jax.ShapeDtypeStruct((B, N, LBL), f32),
        scratch_shapes=[
            pltpu.VMEM((C, N, N), f32),
            pltpu.VMEM((C, N), f32),
            pltpu.VMEM((N, C), f32),
            pltpu.VMEM((N, N), f32),
            pltpu.VMEM((N, N), f32),
            pltpu.VMEM((N, C), f32),
            pltpu.VMEM((C, N), f32),
        ],
    )(x, a, e2, et2, wsi, wsj, bs[None], wsc, wnx, wni, wnj,
      bn[None], Wd, bd[None])
    return out
